# baseline (speedup 1.0000x reference)
import functools
import math

import jax
import jax.numpy as jnp
from jax import lax
from jax.experimental import pallas as pl
from jax.experimental.pallas import tpu as pltpu
from jax.experimental.pallas import tpu_sc as plsc

_D = 64
_SCALE = math.sqrt(float(_D))
_NC, _NS = 2, 16
_NW = _NC * _NS
_CH = 128
_L = 16
_VW = 2 * _D


def _make_embed(B, nch):
    @functools.partial(
        pl.kernel,
        out_type=jax.ShapeDtypeStruct((B, _D), jnp.float32),
        mesh=plsc.VectorSubcoreMesh(core_axis_name="c", subcore_axis_name="s"),
        compiler_params=pltpu.CompilerParams(needs_layout_passes=False),
        scratch_types=[
            pltpu.VMEM((nch, _CH), jnp.int32),
            pltpu.VMEM((_CH,), jnp.int32),
            pltpu.VMEM((_CH,), jnp.int32),
            pltpu.VMEM((_CH, _VW), jnp.float32),
            pltpu.VMEM((_CH, _VW), jnp.float32),
            pltpu.VMEM((_CH, _D), jnp.float32),
            pltpu.SemaphoreType.DMA,
            pltpu.SemaphoreType.DMA,
        ],
    )
    def emb(x_hbm, table, out_hbm, idx_v, vi0, vi1, buf0, buf1, stage, sem0, sem1):
        bpw = nch * _CH
        wid = lax.axis_index("s") * _NC + lax.axis_index("c")
        base = wid * bpw
        pltpu.sync_copy(x_hbm.at[wid], idx_v)
        vis = (vi0, vi1)
        bufs = (buf0, buf1)
        sems = (sem0, sem1)

        def prep(c, vi):
            for j in range(_CH // _L):
                sl = pl.ds(j * _L, _L)
                vi[sl] = lax.shift_right_logical(idx_v[c, sl], 1)

        def extract(c, buf):
            def grp(g, carry):
                rows = g * _L + lax.iota(jnp.int32, _L)
                iv = idx_v[c, pl.ds(g * _L, _L)]
                cb = (iv & 1) * _D
                for d in range(_D):
                    v = plsc.load_gather(buf, [rows, cb + d])
                    plsc.store_scatter(stage, [rows, lax.full((_L,), d, jnp.int32)], v * _SCALE)
                return carry

            lax.fori_loop(0, _CH // _L, grp, 0)

        prep(0, vi0)
        pltpu.async_copy(table.at[vi0], buf0, sem0)

        def outer(i, carry):
            for b in range(2):
                c = i * 2 + b
                nxt = c + 1
                nb = (b + 1) % 2

                @pl.when(nxt < nch)
                def _():
                    prep(nxt, vis[nb])
                    pltpu.async_copy(table.at[vis[nb]], bufs[nb], sems[nb])

                pltpu.make_async_copy(table.at[vis[b]], bufs[b], sems[b]).wait()
                extract(c, bufs[b])
                pltpu.sync_copy(stage, out_hbm.at[pl.ds(base + c * _CH, _CH)])
            return carry

        lax.fori_loop(0, nch // 2, outer, 0)

    return emb


def kernel(x, W):
    S, T = x.shape
    B = S * T
    assert B % (_NW * _CH) == 0
    nch = B // (_NW * _CH)
    idx = x.reshape(_NW, nch, _CH)
    Wv = W.reshape(W.shape[0] // 2, _VW)
    out = _make_embed(B, nch)(idx, Wv)
    return out.reshape(S, T, _D)


# speedup vs baseline: 1.6420x; 1.6420x over previous
import functools
import math

import jax
import jax.numpy as jnp
from jax import lax
from jax.experimental import pallas as pl
from jax.experimental.pallas import tpu as pltpu
from jax.experimental.pallas import tpu_sc as plsc

_D = 64
_SCALE = math.sqrt(float(_D))
_NC, _NS = 2, 16
_NW = _NC * _NS
_CH = 128
_L = 16


def _scale_buf(buf, rows):
    rows_per_iter = 4

    def body(i, carry):
        for rr in range(rows_per_iter):
            r = i * rows_per_iter + rr
            for k in range(_D // _L):
                sl = pl.ds(k * _L, _L)
                buf[r, sl] = buf[r, sl] * _SCALE
        return carry

    lax.fori_loop(0, rows // rows_per_iter, body, 0)


def _make_embed(B, nch):
    @functools.partial(
        pl.kernel,
        out_type=jax.ShapeDtypeStruct((B, _D), jnp.float32),
        mesh=plsc.VectorSubcoreMesh(core_axis_name="c", subcore_axis_name="s"),
        compiler_params=pltpu.CompilerParams(use_tc_tiling_on_sc=False),
        scratch_types=[
            pltpu.VMEM((nch, _CH), jnp.int32),
            pltpu.VMEM((_CH, _D), jnp.float32),
            pltpu.VMEM((_CH, _D), jnp.float32),
            pltpu.SemaphoreType.DMA,
            pltpu.SemaphoreType.DMA,
        ],
    )
    def emb(x_hbm, table, out_hbm, idx_v, buf0, buf1, sem0, sem1):
        bpw = nch * _CH
        wid = lax.axis_index("s") * _NC + lax.axis_index("c")
        base = wid * bpw
        pltpu.sync_copy(x_hbm.at[wid], idx_v)
        bufs = (buf0, buf1)
        sems = (sem0, sem1)

        pltpu.async_copy(table.at[idx_v.at[0]], buf0, sem0)

        def outer(i, carry):
            for b in range(2):
                c = i * 2 + b
                nxt = c + 1
                nb = (b + 1) % 2

                @pl.when(nxt < nch)
                def _():
                    pltpu.async_copy(table.at[idx_v.at[nxt]], bufs[nb], sems[nb])

                pltpu.make_async_copy(table.at[idx_v.at[c]], bufs[b], sems[b]).wait()
                _scale_buf(bufs[b], _CH)
                pltpu.sync_copy(bufs[b], out_hbm.at[pl.ds(base + c * _CH, _CH)])
            return carry

        lax.fori_loop(0, nch // 2, outer, 0)

    return emb


def kernel(x, W):
    S, T = x.shape
    B = S * T
    assert B % (_NW * _CH) == 0
    nch = B // (_NW * _CH)
    idx = x.reshape(_NW, nch, _CH)
    out = _make_embed(B, nch)(idx, W)
    return out.reshape(S, T, _D)
